# baseline (speedup 1.0000x reference)
import functools

import numpy as np
import jax
import jax.numpy as jnp
from jax import lax
from jax.experimental import pallas as pl
from jax.experimental.pallas import tpu as pltpu
from jax.experimental.pallas import tpu_sc as plsc

_B, _T = 16, 32768
_NW = 32
_C = _T // _NW
_P = 1152
_W = _P + _C
_L = 16

_SAMPLE_RATE = 44100
_NUM_VOICES = 4
_RATE = 1.5


def _local_columns() -> np.ndarray:
    base_delay = int(20.0 * _SAMPLE_RATE / 1000)
    range_samples = int(10.0 * _SAMPLE_RATE / 1000 * 0.5)
    tf = np.arange(_T, dtype=np.float64)
    ti = np.arange(_T, dtype=np.int64)
    cols = np.empty((_NUM_VOICES, _T), dtype=np.int32)
    for i in range(_NUM_VOICES):
        phase = (i / _NUM_VOICES + tf * _RATE / _SAMPLE_RATE) % 1.0
        mod = np.sin(2 * np.pi * phase)
        delay = base_delay + np.trunc(mod * range_samples).astype(np.int64)
        delay = np.clip(delay, 1, 2047)
        col = _P + (ti % _C) - delay
        cols[i] = np.where(ti >= delay, col, ti % _L).astype(np.int32)
    return cols


_LIDX = _local_columns()


def _chorus_sc(x, lidx):
    mesh = plsc.VectorSubcoreMesh(core_axis_name="c", subcore_axis_name="s")

    @functools.partial(
        pl.kernel,
        mesh=mesh,
        compiler_params=pltpu.CompilerParams(needs_layout_passes=False),
        out_type=jax.ShapeDtypeStruct((_B, _T), jnp.float32),
        scratch_types=[
            pltpu.VMEM((_B, _W), jnp.float32),
            pltpu.VMEM((_NUM_VOICES * _C,), jnp.int32),
            pltpu.VMEM((_B, _C), jnp.float32),
            pltpu.SemaphoreType.DMA,
        ],
    )
    def k(x_hbm, lidx_hbm, out_hbm, xw, idxv, outv, sem):
        nc = 2
        wid = lax.axis_index("s") * nc + lax.axis_index("c")
        t0 = wid * _C

        h_idx = [
            pltpu.async_copy(lidx_hbm.at[pl.ds(i * _T + t0, _C)],
                             idxv.at[pl.ds(i * _C, _C)], sem)
            for i in range(_NUM_VOICES)
        ]

        zero = jnp.zeros((_L,), jnp.float32)

        @pl.when(wid == 0)
        def _():
            for b in range(_B):
                xw[b, pl.ds(0, _L)] = zero
            pltpu.async_copy(
                x_hbm.at[:, pl.ds(0, _C)],
                xw.at[:, pl.ds(_P, _C)], sem).wait()

        @pl.when(wid == 1)
        def _():
            for b in range(_B):
                xw[b, pl.ds(0, _L)] = zero
            pltpu.async_copy(
                x_hbm.at[:, pl.ds(0, 2 * _C)],
                xw.at[:, pl.ds(_P - _C, 2 * _C)], sem).wait()

        @pl.when(wid >= 2)
        def _():
            pltpu.async_copy(
                x_hbm.at[:, pl.ds(t0 - _P, _W)],
                xw.at[:, pl.ds(0, _W)], sem).wait()

        for h in h_idx:
            h.wait()

        rows = [jnp.full((_L,), b, jnp.int32) for b in range(_B)]

        @plsc.parallel_loop(0, _C // _L, unroll=2)
        def _body(v):
            base = v * _L
            ivs = [idxv[pl.ds(i * _C + base, _L)] for i in range(_NUM_VOICES)]
            for b in range(_B):
                g = plsc.load_gather(xw, [rows[b], ivs[0]])
                for i in range(1, _NUM_VOICES):
                    g = g + plsc.load_gather(xw, [rows[b], ivs[i]])
                dry = xw[b, pl.ds(_P + base, _L)]
                outv[b, pl.ds(base, _L)] = dry * 0.5 + g * 0.125

        pltpu.async_copy(outv, out_hbm.at[:, pl.ds(t0, _C)], sem).wait()

    return k(x, jnp.asarray(lidx).reshape(-1))


def kernel(x):
    return _chorus_sc(x, _LIDX)


# speedup vs baseline: 1.1327x; 1.1327x over previous
import functools

import numpy as np
import jax
import jax.numpy as jnp
from jax import lax
from jax.experimental import pallas as pl
from jax.experimental.pallas import tpu as pltpu
from jax.experimental.pallas import tpu_sc as plsc

_B, _T = 16, 32768
_NW = 32
_C = _T // _NW
_P = 1152
_W = _P + _C
_L = 16

_SAMPLE_RATE = 44100
_NUM_VOICES = 4
_RATE = 1.5


def _local_columns() -> np.ndarray:
    base_delay = int(20.0 * _SAMPLE_RATE / 1000)
    range_samples = int(10.0 * _SAMPLE_RATE / 1000 * 0.5)
    tf = np.arange(_T, dtype=np.float64)
    ti = np.arange(_T, dtype=np.int64)
    cols = np.empty((_NUM_VOICES, _T), dtype=np.int32)
    for i in range(_NUM_VOICES):
        phase = (i / _NUM_VOICES + tf * _RATE / _SAMPLE_RATE) % 1.0
        mod = np.sin(2 * np.pi * phase)
        delay = base_delay + np.trunc(mod * range_samples).astype(np.int64)
        delay = np.clip(delay, 1, 2047)
        col = _P + (ti % _C) - delay
        cols[i] = np.where(ti >= delay, col, ti % _L).astype(np.int32)
    return cols


_LIDX = _local_columns()


def _chorus_sc(x, lidx):
    mesh = plsc.VectorSubcoreMesh(core_axis_name="c", subcore_axis_name="s")

    @functools.partial(
        pl.kernel,
        mesh=mesh,
        compiler_params=pltpu.CompilerParams(needs_layout_passes=False),
        out_type=jax.ShapeDtypeStruct((_B, _T), jnp.float32),
        scratch_types=[
            pltpu.VMEM((_B, _W), jnp.float32),
            pltpu.VMEM((_NUM_VOICES * _C,), jnp.int32),
            pltpu.VMEM((_B, _C), jnp.float32),
            pltpu.SemaphoreType.DMA,
            pltpu.SemaphoreType.DMA,
            pltpu.SemaphoreType.DMA,
        ],
    )
    def k(x_hbm, lidx_hbm, out_hbm, xw, idxv, outv, sem, semb, semo):
        nc = 2
        wid = lax.axis_index("s") * nc + lax.axis_index("c")
        t0 = wid * _C
        _H = _C // 2
        _A = _P + _H

        h_idx = [
            pltpu.async_copy(lidx_hbm.at[pl.ds(i * _T + t0, _C)],
                             idxv.at[pl.ds(i * _C, _C)], sem)
            for i in range(_NUM_VOICES)
        ]

        zero = jnp.zeros((_L,), jnp.float32)

        @pl.when(wid == 0)
        def _():
            for b in range(_B):
                xw[b, pl.ds(0, _L)] = zero
            pltpu.async_copy(x_hbm.at[:, pl.ds(_H, _H)],
                             xw.at[:, pl.ds(_A, _H)], semb)
            pltpu.async_copy(x_hbm.at[:, pl.ds(0, _H)],
                             xw.at[:, pl.ds(_P, _H)], sem).wait()

        @pl.when(wid == 1)
        def _():
            for b in range(_B):
                xw[b, pl.ds(0, _L)] = zero
            pltpu.async_copy(x_hbm.at[:, pl.ds(3 * _H, _H)],
                             xw.at[:, pl.ds(_A, _H)], semb)
            pltpu.async_copy(x_hbm.at[:, pl.ds(0, 3 * _H)],
                             xw.at[:, pl.ds(_P - _C, 3 * _H)], sem).wait()

        @pl.when(wid >= 2)
        def _():
            pltpu.async_copy(x_hbm.at[:, pl.ds(t0 + _H, _H)],
                             xw.at[:, pl.ds(_A, _H)], semb)
            pltpu.async_copy(x_hbm.at[:, pl.ds(t0 - _P, _A)],
                             xw.at[:, pl.ds(0, _A)], sem).wait()

        for h in h_idx:
            h.wait()

        rows = [jnp.full((_L,), b, jnp.int32) for b in range(_B)]

        def make_body(v):
            base = v * _L
            ivs = [idxv[pl.ds(i * _C + base, _L)] for i in range(_NUM_VOICES)]
            for b in range(_B):
                g = plsc.load_gather(xw, [rows[b], ivs[0]])
                for i in range(1, _NUM_VOICES):
                    g = g + plsc.load_gather(xw, [rows[b], ivs[i]])
                dry = xw[b, pl.ds(_P + base, _L)]
                outv[b, pl.ds(base, _L)] = dry * 0.5 + g * 0.125

        plsc.parallel_loop(0, _H // _L)(make_body)

        h_out1 = pltpu.async_copy(outv.at[:, pl.ds(0, _H)],
                                  out_hbm.at[:, pl.ds(t0, _H)], semo)

        pltpu.make_async_copy(x_hbm.at[:, pl.ds(0, _H)],
                              xw.at[:, pl.ds(_A, _H)], semb).wait()

        plsc.parallel_loop(_H // _L, _C // _L)(make_body)

        h_out2 = pltpu.async_copy(outv.at[:, pl.ds(_H, _H)],
                                  out_hbm.at[:, pl.ds(t0 + _H, _H)], semo)
        h_out1.wait()
        h_out2.wait()

    return k(x, jnp.asarray(lidx).reshape(-1))


def kernel(x):
    return _chorus_sc(x, _LIDX)


# speedup vs baseline: 1.1909x; 1.0513x over previous
import functools

import numpy as np
import jax
import jax.numpy as jnp
from jax import lax
from jax.experimental import pallas as pl
from jax.experimental.pallas import tpu as pltpu
from jax.experimental.pallas import tpu_sc as plsc

_B, _T = 16, 32768
_NW = 32
_C = _T // _NW
_P = 1152
_W = _P + _C
_L = 16

_SAMPLE_RATE = 44100
_NUM_VOICES = 4
_RATE = 1.5


def _packed_delays() -> np.ndarray:
    base_delay = int(20.0 * _SAMPLE_RATE / 1000)
    range_samples = int(10.0 * _SAMPLE_RATE / 1000 * 0.5)
    tf = np.arange(_T, dtype=np.float64)
    ti = np.arange(_T, dtype=np.int64)
    deltas = np.empty((_NUM_VOICES, _T), dtype=np.int64)
    for i in range(_NUM_VOICES):
        phase = (i / _NUM_VOICES + tf * _RATE / _SAMPLE_RATE) % 1.0
        mod = np.sin(2 * np.pi * phase)
        delay = base_delay + np.trunc(mod * range_samples).astype(np.int64)
        delay = np.clip(delay, 1, 2047)
        deltas[i] = np.where(ti >= delay, delay, _P + (ti % _C) - (ti % _L))
    assert deltas.min() >= 0 and deltas.max() < 65536
    packed = np.empty((2, _T), dtype=np.int32)
    packed[0] = (deltas[0] | (deltas[1] << 16)).astype(np.uint32).view(np.int32)
    packed[1] = (deltas[2] | (deltas[3] << 16)).astype(np.uint32).view(np.int32)
    return packed


_LIDX = _packed_delays()


def _chorus_sc(x, lidx):
    mesh = plsc.VectorSubcoreMesh(core_axis_name="c", subcore_axis_name="s")

    @functools.partial(
        pl.kernel,
        mesh=mesh,
        compiler_params=pltpu.CompilerParams(needs_layout_passes=False),
        out_type=jax.ShapeDtypeStruct((_B, _T), jnp.float32),
        scratch_types=[
            pltpu.VMEM((_B, _W), jnp.float32),
            pltpu.VMEM((2 * _C,), jnp.int32),
            pltpu.VMEM((_B, _C), jnp.float32),
            pltpu.SemaphoreType.DMA,
        ],
    )
    def k(x_hbm, lidx_hbm, out_hbm, xw, idxv, outv, sem):
        nc = 2
        wid = lax.axis_index("s") * nc + lax.axis_index("c")
        t0 = wid * _C

        h_idx = [
            pltpu.async_copy(lidx_hbm.at[pl.ds(i * _T + t0, _C)],
                             idxv.at[pl.ds(i * _C, _C)], sem)
            for i in range(2)
        ]

        zero = jnp.zeros((_L,), jnp.float32)

        @pl.when(wid == 0)
        def _():
            for b in range(_B):
                xw[b, pl.ds(0, _L)] = zero
            pltpu.async_copy(
                x_hbm.at[:, pl.ds(0, _C)],
                xw.at[:, pl.ds(_P, _C)], sem).wait()

        @pl.when(wid == 1)
        def _():
            for b in range(_B):
                xw[b, pl.ds(0, _L)] = zero
            pltpu.async_copy(
                x_hbm.at[:, pl.ds(0, 2 * _C)],
                xw.at[:, pl.ds(_P - _C, 2 * _C)], sem).wait()

        @pl.when(wid >= 2)
        def _():
            pltpu.async_copy(
                x_hbm.at[:, pl.ds(t0 - _P, _W)],
                xw.at[:, pl.ds(0, _W)], sem).wait()

        for h in h_idx:
            h.wait()

        rows = [jnp.full((_L,), b, jnp.int32) for b in range(_B)]
        lane = lax.iota(jnp.int32, _L)
        mask16 = jnp.full((_L,), 0xFFFF, jnp.int32)
        sh16 = jnp.full((_L,), 16, jnp.int32)

        @plsc.parallel_loop(0, _C // _L)
        def _body(v):
            base = v * _L
            tvec = lane + (_P + base)
            w01 = idxv[pl.ds(base, _L)]
            w23 = idxv[pl.ds(_C + base, _L)]
            ivs = [tvec - (w01 & mask16),
                   tvec - lax.shift_right_logical(w01, sh16),
                   tvec - (w23 & mask16),
                   tvec - lax.shift_right_logical(w23, sh16)]
            for b in range(_B):
                g = plsc.load_gather(xw, [rows[b], ivs[0]])
                for i in range(1, _NUM_VOICES):
                    g = g + plsc.load_gather(xw, [rows[b], ivs[i]])
                dry = xw[b, pl.ds(_P + base, _L)]
                outv[b, pl.ds(base, _L)] = dry * 0.5 + g * 0.125

        pltpu.async_copy(outv, out_hbm.at[:, pl.ds(t0, _C)], sem).wait()

    return k(x, jnp.asarray(lidx).reshape(-1))


def kernel(x):
    return _chorus_sc(x, _LIDX)
